# baseline (speedup 1.0000x reference)
import jax
import jax.numpy as jnp
from jax import lax
from jax.experimental import pallas as pl
from jax.experimental.pallas import tpu as pltpu
from jax.experimental.pallas import tpu_sc as plsc

NC = 2
NS = 16
NW = NC * NS
CHUNK = 128
PADW = 128


def _make_body(B, L, D, BPW, NCHUNK, NSTREAM):
    def body(idx_hbm, table_hbm, out_hbm, idx_raw_v, idx_t_v, acc_v, sem):
        lane = jax.lax.iota(jnp.int32, 16)
        wid = lax.axis_index("s") * NC + lax.axis_index("c")
        pltpu.sync_copy(idx_hbm.at[wid], idx_raw_v)

        zeros = jnp.zeros((16,), jnp.float32)

        def zero_row(b, carry):
            for h in range(D // 16):
                acc_v[b, pl.ds(h * 16, 16)] = zeros
            return carry

        lax.fori_loop(0, BPW, zero_row, 0)

        def transpose_row(r, carry):
            l = r // NCHUNK
            c = lax.rem(r, NCHUNK)
            lvec = jnp.broadcast_to(l, (16,)).astype(jnp.int32)
            base = c * CHUNK
            for g in range(CHUNK // 16):
                rows = base + g * 16 + lane
                v = plsc.load_gather(idx_raw_v, [rows, lvec])
                idx_t_v[r, pl.ds(g * 16, 16)] = v
            return carry

        lax.fori_loop(0, NSTREAM, transpose_row, 0)

        def fire(r, carry):
            c = lax.rem(r, NCHUNK)
            pltpu.async_copy(
                table_hbm.at[idx_t_v.at[r]],
                acc_v.at[pl.ds(c * CHUNK, CHUNK)],
                sem,
                add=True,
            )
            return carry

        lax.fori_loop(0, NSTREAM, fire, 0)

        def drain(r, carry):
            pltpu.make_async_copy(
                table_hbm.at[idx_t_v.at[0]],
                acc_v.at[pl.ds(0, CHUNK)],
                sem,
            ).wait()
            return carry

        lax.fori_loop(0, NSTREAM, drain, 0)

        scale = jnp.float32(1.0 / L)

        def scale_row(b, carry):
            for h in range(D // 16):
                acc_v[b, pl.ds(h * 16, 16)] = acc_v[b, pl.ds(h * 16, 16)] * scale
            return carry

        lax.fori_loop(0, BPW, scale_row, 0)
        pltpu.sync_copy(
            acc_v.at[:, pl.ds(0, D)], out_hbm.at[pl.ds(wid * BPW, BPW)]
        )

    return body


def kernel(inputs, table):
    B, L = inputs.shape
    V, D = table.shape
    BPW = B // NW
    NCHUNK = BPW // CHUNK
    NSTREAM = L * NCHUNK

    idx = inputs.astype(jnp.int32).reshape(NW, BPW, L)
    table_p = jnp.pad(table, ((0, 0), (0, PADW - D)))

    mesh = plsc.VectorSubcoreMesh(
        core_axis_name="c", subcore_axis_name="s", num_cores=NC, num_subcores=NS
    )
    f = pl.kernel(
        _make_body(B, L, D, BPW, NCHUNK, NSTREAM),
        out_type=jax.ShapeDtypeStruct((B, D), jnp.float32),
        mesh=mesh,
        scratch_types=[
            pltpu.VMEM((BPW, L), jnp.int32),
            pltpu.VMEM((NSTREAM, CHUNK), jnp.int32),
            pltpu.VMEM((BPW, PADW), jnp.float32),
            pltpu.SemaphoreType.DMA,
        ],
        compiler_params=pltpu.CompilerParams(
            use_tc_tiling_on_sc=False, needs_layout_passes=False
        ),
    )
    return f(idx, table_p)


# speedup vs baseline: 1.8507x; 1.8507x over previous
import jax
import jax.numpy as jnp
from jax import lax
from jax.experimental import pallas as pl
from jax.experimental.pallas import tpu as pltpu
from jax.experimental.pallas import tpu_sc as plsc

NC = 2
NS = 16
NW = NC * NS
CHUNK = 128
VB = 8192


QW = 2048


def _tc_body(in_ref, out_ref):
    x = in_ref[...]
    parts = [
        jnp.transpose(x[:, q * QW : (q + 1) * QW]) for q in range(VB // QW)
    ]
    out_ref[...] = jnp.concatenate(parts, axis=1)


def _relayout(table_t):
    D, V = table_t.shape
    grid = (V + VB - 1) // VB
    return pl.pallas_call(
        _tc_body,
        grid=(grid,),
        in_specs=[pl.BlockSpec((D, VB), lambda i: (0, i))],
        out_specs=pl.BlockSpec((QW, 128), lambda i: (i, 0)),
        out_shape=jax.ShapeDtypeStruct((grid * QW, 128), jnp.float32),
    )(table_t)


def _make_body(B, L, D, BPW, NCHUNK, NSTREAM):
    def body(idx_hbm, table_hbm, out_hbm, idx_raw_v, idx_t_v, acc_v, sem):
        lane = jax.lax.iota(jnp.int32, 16)
        wid = lax.axis_index("s") * NC + lax.axis_index("c")
        pltpu.sync_copy(idx_hbm.at[wid], idx_raw_v)

        zeros = jnp.zeros((16,), jnp.float32)

        def zero_row(b, carry):
            for h in range(D // 16):
                acc_v[b, pl.ds(h * 16, 16)] = zeros
            return carry

        lax.fori_loop(0, BPW, zero_row, 0)

        def transpose_row(r, carry):
            l = r // NCHUNK
            c = lax.rem(r, NCHUNK)
            lvec = jnp.broadcast_to(l, (16,)).astype(jnp.int32)
            base = c * CHUNK
            for g in range(CHUNK // 16):
                rows = base + g * 16 + lane
                v = plsc.load_gather(idx_raw_v, [rows, lvec])
                j = lax.rem(v, VB)
                pi = (v - j) + (lax.rem(j, QW) * 4) + (j // QW)
                idx_t_v[r, pl.ds(g * 16, 16)] = pi
            return carry

        lax.fori_loop(0, NSTREAM, transpose_row, 0)

        def fire(r, carry):
            c = lax.rem(r, NCHUNK)
            pltpu.async_copy(
                table_hbm.at[idx_t_v.at[r]],
                acc_v.at[pl.ds(c * CHUNK, CHUNK)],
                sem,
                add=True,
            )
            return carry

        lax.fori_loop(0, NSTREAM, fire, 0)

        def drain(r, carry):
            pltpu.make_async_copy(
                table_hbm.at[idx_t_v.at[0]],
                acc_v.at[pl.ds(0, CHUNK)],
                sem,
            ).wait()
            return carry

        lax.fori_loop(0, NSTREAM, drain, 0)

        scale = jnp.float32(1.0 / L)

        def scale_row(b, carry):
            for h in range(D // 16):
                acc_v[b, pl.ds(h * 16, 16)] = acc_v[b, pl.ds(h * 16, 16)] * scale
            return carry

        lax.fori_loop(0, BPW, scale_row, 0)
        pltpu.sync_copy(acc_v, out_hbm.at[pl.ds(wid * BPW, BPW)])

    return body


def kernel(inputs, table):
    B, L = inputs.shape
    V, D = table.shape
    BPW = B // NW
    NCHUNK = BPW // CHUNK
    NSTREAM = L * NCHUNK

    idx = inputs.astype(jnp.int32).reshape(NW, BPW, L)

    table_q = _relayout(table.T)
    table_rm = table_q.reshape(table_q.shape[0] * (128 // D), D)

    mesh = plsc.VectorSubcoreMesh(
        core_axis_name="c", subcore_axis_name="s", num_cores=NC, num_subcores=NS
    )
    f = pl.kernel(
        _make_body(B, L, D, BPW, NCHUNK, NSTREAM),
        out_type=jax.ShapeDtypeStruct((B, D), jnp.float32),
        mesh=mesh,
        scratch_types=[
            pltpu.VMEM((BPW, L), jnp.int32),
            pltpu.VMEM((NSTREAM, CHUNK), jnp.int32),
            pltpu.VMEM((BPW, D), jnp.float32),
            pltpu.SemaphoreType.DMA,
        ],
        compiler_params=pltpu.CompilerParams(
            use_tc_tiling_on_sc=False, needs_layout_passes=False
        ),
    )
    return f(idx, table_rm)
